# baseline (speedup 1.0000x reference)
import functools

import jax
import jax.numpy as jnp
from jax import lax
from jax.experimental import pallas as pl
from jax.experimental.pallas import tpu as pltpu
from jax.experimental.pallas import tpu_sc as plsc

B = 1024
L_CTX = 20
V = 100000
D = 32
H = 1024
N_IDX = B * L_CTX
NW = 32
B_PER_W = N_IDX // NW
CHUNK = 128
N_CHUNK = B_PER_W // CHUNK

VT = 2048
N_VT = (V + VT - 1) // VT


def _gather_body(idx_hbm, table_hbm, out_hbm, idx_v, rows_v, sem):
    wid = lax.axis_index("s") * 2 + lax.axis_index("c")
    pltpu.sync_copy(idx_hbm.at[wid], idx_v)
    copies = []
    for j in range(N_CHUNK):
        copies.append(
            pltpu.async_copy(
                table_hbm.at[idx_v.at[j]],
                rows_v.at[pl.ds(j * CHUNK, CHUNK)],
                sem,
            )
        )
    for c in copies:
        c.wait()
    pltpu.sync_copy(rows_v, out_hbm.at[pl.ds(wid * B_PER_W, B_PER_W)])


@functools.cache
def _sc_gather():
    return pl.kernel(
        _gather_body,
        out_type=jax.ShapeDtypeStruct((N_IDX, D), jnp.float32),
        mesh=plsc.VectorSubcoreMesh(core_axis_name="c", subcore_axis_name="s"),
        scratch_types=[
            pltpu.VMEM((N_CHUNK, CHUNK), jnp.int32),
            pltpu.VMEM((B_PER_W, D), jnp.float32),
            pltpu.SemaphoreType.DMA,
        ],
        compiler_params=pltpu.CompilerParams(use_tc_tiling_on_sc=False),
    )


def _mlp_body(hg_ref, w1_ref, b1_ref, w2_ref, b2_ref, out_ref, h1_ref):
    @pl.when(pl.program_id(0) == 0)
    def _():
        h1 = jnp.dot(hg_ref[...], w1_ref[...], preferred_element_type=jnp.float32)
        h1 = jnp.maximum(h1 + b1_ref[...], 0.0)
        h1_ref[...] = h1.astype(jnp.bfloat16)

    w2 = w2_ref[...].astype(jnp.bfloat16)
    acc = jnp.dot(h1_ref[...], w2, preferred_element_type=jnp.float32)
    out_ref[...] = acc + b2_ref[...]


_mlp = pl.pallas_call(
    _mlp_body,
    grid=(N_VT,),
    in_specs=[
        pl.BlockSpec((B, L_CTX * D), lambda j: (0, 0)),
        pl.BlockSpec((L_CTX * D, H), lambda j: (0, 0)),
        pl.BlockSpec((1, H), lambda j: (0, 0)),
        pl.BlockSpec((H, VT), lambda j: (0, j)),
        pl.BlockSpec((1, VT), lambda j: (0, j)),
    ],
    out_specs=pl.BlockSpec((B, VT), lambda j: (0, j)),
    out_shape=jax.ShapeDtypeStruct((B, V), jnp.float32),
    scratch_shapes=[pltpu.VMEM((B, H), jnp.bfloat16)],
    compiler_params=pltpu.CompilerParams(
        dimension_semantics=("arbitrary",),
    ),
)


@jax.jit
def kernel(x, emb, W1, b1, W2, b2):
    idx = x.reshape(NW, N_CHUNK, CHUNK).astype(jnp.int32)
    rows = _sc_gather()(idx, emb)
    hg = rows.reshape(B, L_CTX * D)
    return _mlp(hg, W1, b1.reshape(1, H), W2, b2.reshape(1, V))
